# baseline (speedup 1.0000x reference)
import jax
import jax.numpy as jnp
from jax import lax
from jax.experimental import pallas as pl
from jax.experimental.pallas import tpu as pltpu
from jax.experimental.pallas import tpu_sc as plsc

_B = 128
_N_ATOM = 100000
_N_X = 1024
_N_Y = 32768
_NW = 32
_YPW = _N_Y // _NW
_NVY = _YPW // 16
_APW = _N_ATOM // _NW
_SEGW = 3136
_NVSEG = _SEGW // 16


def _sc_body(xf_h, xb_h, y3_h, yb_h, seg_h, aty_h, y0c_h,
             colmin_out, part_out,
             xf_v, xb_v, xs_v, xe_v,
             yt_v, yb_v, colmin_v, seg_v, aty_v, y016_v,
             part_v, mrg_v, cm_v, sem,
             colmin_sh):
    cid = lax.axis_index("c")
    sid = lax.axis_index("s")
    wid = cid * 16 + sid

    alo = wid * _APW
    aw = pl.multiple_of(jnp.minimum(alo & -8, _N_ATOM - _SEGW), 8)

    cps = [
        pltpu.async_copy(xf_h, xf_v, sem),
        pltpu.async_copy(xb_h, xb_v, sem),
        pltpu.async_copy(y3_h.at[wid], yt_v, sem),
        pltpu.async_copy(yb_h.at[pl.ds(wid * _YPW, _YPW)], yb_v, sem),
        pltpu.async_copy(seg_h.at[pl.ds(aw, _SEGW)], seg_v, sem),
        pltpu.async_copy(aty_h.at[pl.ds(aw, _SEGW)], aty_v, sem),
        pltpu.async_copy(y0c_h, y016_v, sem),
    ]

    inf16 = jnp.full((16,), jnp.inf, jnp.float32)
    zero16 = jnp.zeros((16,), jnp.float32)
    lane = lax.iota(jnp.int32, 16)
    lane0 = lane == 0

    @plsc.parallel_loop(0, _N_X // 16, unroll=4)
    def _init_body(i):
        colmin_v[pl.ds(i * 16, 16)] = inf16

    for cp in cps:
        cp.wait()

    @plsc.parallel_loop(0, _B // 16, unroll=2)
    def bs_body(t):
        b = lane + t * 16
        lo = jnp.zeros((16,), jnp.int32)
        hi = jnp.full((16,), _N_X, jnp.int32)
        lo2 = jnp.zeros((16,), jnp.int32)
        hi2 = jnp.full((16,), _N_X, jnp.int32)
        for _unused in range(11):
            mid = jnp.minimum((lo + hi) >> 1, _N_X - 1)
            v = plsc.load_gather(xb_v, [mid])
            p = v < b
            lo = jnp.where(p, mid + 1, lo)
            hi = jnp.where(p, hi, mid)
            mid2 = jnp.minimum((lo2 + hi2) >> 1, _N_X - 1)
            v2 = plsc.load_gather(xb_v, [mid2])
            p2 = v2 <= b
            lo2 = jnp.where(p2, mid2 + 1, lo2)
            hi2 = jnp.where(p2, hi2, mid2)
        xs_v[pl.ds(t * 16, 16)] = lo
        xe_v[pl.ds(t * 16, 16)] = lo2

    def yblk(k, hub16):
        yb16 = yb_v[pl.ds(k * 16, 16)]
        y0 = yt_v[0, pl.ds(k * 16, 16)]
        y1 = yt_v[1, pl.ds(k * 16, 16)]
        y2 = yt_v[2, pl.ds(k * 16, 16)]
        s = plsc.load_gather(xs_v, [yb16])
        e = plsc.load_gather(xe_v, [yb16])
        jlo = jnp.min(s)
        jhi = jnp.max(e)

        @plsc.parallel_loop(jlo, jhi, unroll=8, carry=(inf16, jnp.zeros((16,), jnp.int32)))
        def inner(j, carry):
            best, bidx = carry
            j3 = jnp.full((16,), j * 3, jnp.int32)
            a0 = plsc.load_gather(xf_v, [j3])
            a1 = plsc.load_gather(xf_v, [j3 + 1])
            a2 = plsc.load_gather(xf_v, [j3 + 2])
            d0 = y0 - a0
            d1 = y1 - a1
            d2c = y2 - a2
            dd = d0 * d0 + d1 * d1 + d2c * d2c
            inb = (s <= j) & (j < e)
            ddm = jnp.where(inb, dd, jnp.inf)
            upd = ddm < best
            best = jnp.where(upd, ddm, best)
            bidx = jnp.where(upd, jnp.full((16,), j, jnp.int32), bidx)
            cm = jnp.min(ddm)
            j16 = jnp.full((16,), j, jnp.int32)
            cur = plsc.load_gather(colmin_v, [j16])
            plsc.store_scatter(colmin_v, [j16], jnp.minimum(cur, cm),
                               mask=lane0)
            return best, bidx

        best, bidx = inner
        b3 = bidx * 3
        xg0 = plsc.load_gather(xf_v, [b3])
        xg1 = plsc.load_gather(xf_v, [b3 + 1])
        xg2 = plsc.load_gather(xf_v, [b3 + 2])
        for yc, xg in ((y0, xg0), (y1, xg1), (y2, xg2)):
            err = yc - xg
            a = jnp.abs(err)
            hub16 = hub16 + jnp.where(a < 1.0, 0.5 * err * err, a - 0.5)
        return hub16

    hub16 = lax.fori_loop(0, _NVY, yblk, zero16)

    @plsc.parallel_loop(0, _NVSEG, unroll=4,
                        carry=(zero16, zero16, zero16))
    def dbody(k, carry):
        sp, spt, st = carry
        g = aw + k * 16 + lane
        m = (g >= alo) & (g < alo + _APW)
        z = seg_v[pl.ds(k * 16, 16)]
        p = 1.0 / (1.0 + jnp.exp(-z))
        t = aty_v[pl.ds(k * 16, 16)]
        return (sp + jnp.where(m, p, 0.0),
                spt + jnp.where(m, p * t, 0.0),
                st + jnp.where(m, t, 0.0))
    sp, spt, st = dbody

    part_v[pl.ds(0, 16)] = hub16
    part_v[pl.ds(16, 16)] = sp
    part_v[pl.ds(32, 16)] = spt
    part_v[pl.ds(48, 16)] = st
    pltpu.sync_copy(part_v, part_out.at[pl.ds(wid * 64, 64)])
    pltpu.sync_copy(colmin_v, colmin_sh.at[sid])
    plsc.subcore_barrier()

    for r in range(16):
        pltpu.sync_copy(colmin_sh.at[r, pl.ds(sid * 64, 64)],
                        mrg_v.at[pl.ds(r * 64, 64)])

    def mbody(r, carry):
        c0, c1, c2, c3 = carry
        c0 = jnp.minimum(c0, mrg_v[pl.ds(r * 64, 16)])
        c1 = jnp.minimum(c1, mrg_v[pl.ds(r * 64 + 16, 16)])
        c2 = jnp.minimum(c2, mrg_v[pl.ds(r * 64 + 32, 16)])
        c3 = jnp.minimum(c3, mrg_v[pl.ds(r * 64 + 48, 16)])
        return c0, c1, c2, c3
    cs = lax.fori_loop(0, 16, mbody, (inf16, inf16, inf16, inf16))

    y0c0 = plsc.load_gather(y016_v, [jnp.zeros((16,), jnp.int32)])
    y0c1 = plsc.load_gather(y016_v, [jnp.full((16,), 1, jnp.int32)])
    y0c2 = plsc.load_gather(y016_v, [jnp.full((16,), 2, jnp.int32)])
    for q in range(4):
        ci3 = (sid * 64 + q * 16 + lane) * 3
        g0 = plsc.load_gather(xf_v, [ci3])
        g1 = plsc.load_gather(xf_v, [ci3 + 1])
        g2 = plsc.load_gather(xf_v, [ci3 + 2])
        e0 = g0 - y0c0
        e1 = g1 - y0c1
        e2 = g2 - y0c2
        d2y0 = e0 * e0 + e1 * e1 + e2 * e2
        cq = jnp.where(jnp.isinf(cs[q]), d2y0, cs[q])
        cm_v[pl.ds(q * 16, 16)] = cq
    pltpu.sync_copy(cm_v, colmin_out.at[pl.ds(cid * _N_X + sid * 64, 64)])


def _fin_body(cm2, p2r, conf, out):
    cm = jnp.minimum(cm2[pl.ds(0, _N_X)], cm2[pl.ds(_N_X, _N_X)])
    dist = jnp.sqrt(cm)
    dc = conf[...] - dist.reshape(_N_X, 1)
    conf_loss = jnp.sum(dc * dc) / _N_X
    p2 = p2r[...]
    q = (lax.broadcasted_iota(jnp.int32, (_NW * 64,), 0) // 16) % 4
    hub = jnp.sum(jnp.where(q == 0, p2, 0.0))
    sp = jnp.sum(jnp.where(q == 1, p2, 0.0))
    spt = jnp.sum(jnp.where(q == 2, p2, 0.0))
    st = jnp.sum(jnp.where(q == 3, p2, 0.0))
    eps = 1e-6
    dice = 1.0 - (2.0 * spt + eps) / (sp + st + eps)
    out[...] = (hub / (_N_Y * 3.0) + dice + conf_loss).reshape(1, 1)


@jax.jit
def kernel(pred_seg, atom_y, pred_pos_global_node, bindingsite_center,
           preds_confidence, x_batch, y_batch):
    xf = pred_pos_global_node.reshape(-1)
    yt = bindingsite_center.T
    y3 = yt.reshape(3, _NW, _YPW).transpose(1, 0, 2)
    y0c16 = jnp.pad(yt[:, 0], (0, 13))
    seg = pred_seg.reshape(-1)

    f32 = jnp.float32
    i32 = jnp.int32
    sc = pl.kernel(
        _sc_body,
        out_type=(jax.ShapeDtypeStruct((2 * _N_X,), f32),
                  jax.ShapeDtypeStruct((_NW * 64,), f32)),
        mesh=plsc.VectorSubcoreMesh(core_axis_name="c", subcore_axis_name="s"),
        compiler_params=pltpu.CompilerParams(needs_layout_passes=False),
        scratch_types=[
            pltpu.VMEM((3 * _N_X,), f32),
            pltpu.VMEM((_N_X,), i32),
            pltpu.VMEM((_B,), i32),
            pltpu.VMEM((_B,), i32),
            pltpu.VMEM((3, _YPW), f32),
            pltpu.VMEM((_YPW,), i32),
            pltpu.VMEM((_N_X,), f32),
            pltpu.VMEM((_SEGW,), f32),
            pltpu.VMEM((_SEGW,), f32),
            pltpu.VMEM((16,), f32),
            pltpu.VMEM((64,), f32),
            pltpu.VMEM((1024,), f32),
            pltpu.VMEM((64,), f32),
            pltpu.SemaphoreType.DMA,
            pltpu.VMEM_SHARED((16, _N_X), f32),
        ],
    )
    colmin2, part2 = sc(xf, x_batch, y3, y_batch, seg, atom_y, y0c16)

    out = pl.pallas_call(
        _fin_body,
        out_shape=jax.ShapeDtypeStruct((1, 1), f32),
    )(colmin2, part2, preds_confidence)
    return out[0, 0]


# speedup vs baseline: 1.0559x; 1.0559x over previous
import jax
import jax.numpy as jnp
from jax import lax
from jax.experimental import pallas as pl
from jax.experimental.pallas import tpu as pltpu
from jax.experimental.pallas import tpu_sc as plsc

_B = 128
_N_ATOM = 100000
_N_X = 1024
_N_Y = 32768
_NW = 32
_YPW = _N_Y // _NW
_NVY = _YPW // 16
_APW = _N_ATOM // _NW
_SEGW = 3136
_NVSEG = _SEGW // 16


def _sc_body(xf_h, xb_h, y3_h, yb_h, seg_h, aty_h, y0c_h,
             colmin_out, part_out,
             xf_v, xb_v, xs_v, xe_v,
             yt_v, yb_v, colmin_v, seg_v, aty_v, y016_v,
             part_v, mrg_v, cm_v, sem,
             colmin_sh):
    cid = lax.axis_index("c")
    sid = lax.axis_index("s")
    wid = cid * 16 + sid

    alo = wid * _APW
    aw = pl.multiple_of(jnp.minimum(alo & -8, _N_ATOM - _SEGW), 8)

    cps = [
        pltpu.async_copy(xf_h, xf_v, sem),
        pltpu.async_copy(xb_h, xb_v, sem),
        pltpu.async_copy(y3_h.at[wid], yt_v, sem),
        pltpu.async_copy(yb_h.at[pl.ds(wid * _YPW, _YPW)], yb_v, sem),
        pltpu.async_copy(seg_h.at[pl.ds(aw, _SEGW)], seg_v, sem),
        pltpu.async_copy(aty_h.at[pl.ds(aw, _SEGW)], aty_v, sem),
        pltpu.async_copy(y0c_h, y016_v, sem),
    ]

    inf16 = jnp.full((16,), jnp.inf, jnp.float32)
    zero16 = jnp.zeros((16,), jnp.float32)
    lane = lax.iota(jnp.int32, 16)
    lane0 = lane == 0

    @plsc.parallel_loop(0, _N_X // 16, unroll=4)
    def _init_body(i):
        colmin_v[pl.ds(i * 16, 16)] = inf16

    for cp in cps:
        cp.wait()

    @plsc.parallel_loop(0, _B // 16, unroll=2)
    def bs_body(t):
        b = lane + t * 16
        lo = jnp.zeros((16,), jnp.int32)
        hi = jnp.full((16,), _N_X, jnp.int32)
        lo2 = jnp.zeros((16,), jnp.int32)
        hi2 = jnp.full((16,), _N_X, jnp.int32)
        for _unused in range(11):
            mid = jnp.minimum((lo + hi) >> 1, _N_X - 1)
            v = plsc.load_gather(xb_v, [mid])
            p = v < b
            lo = jnp.where(p, mid + 1, lo)
            hi = jnp.where(p, hi, mid)
            mid2 = jnp.minimum((lo2 + hi2) >> 1, _N_X - 1)
            v2 = plsc.load_gather(xb_v, [mid2])
            p2 = v2 <= b
            lo2 = jnp.where(p2, mid2 + 1, lo2)
            hi2 = jnp.where(p2, hi2, mid2)
        xs_v[pl.ds(t * 16, 16)] = lo
        xe_v[pl.ds(t * 16, 16)] = lo2

    def yblk(k, hub16):
        yb16 = yb_v[pl.ds(k * 16, 16)]
        y0 = yt_v[0, pl.ds(k * 16, 16)]
        y1 = yt_v[1, pl.ds(k * 16, 16)]
        y2 = yt_v[2, pl.ds(k * 16, 16)]
        s = plsc.load_gather(xs_v, [yb16])
        e = plsc.load_gather(xe_v, [yb16])
        jlo = jnp.min(s)
        jhi = jnp.max(e)

        @plsc.parallel_loop(jlo, jhi, unroll=4, carry=(inf16, jnp.zeros((16,), jnp.int32)))
        def inner(j, carry):
            best, bidx = carry
            j3 = jnp.full((16,), j * 3, jnp.int32)
            a0 = plsc.load_gather(xf_v, [j3])
            a1 = plsc.load_gather(xf_v, [j3 + 1])
            a2 = plsc.load_gather(xf_v, [j3 + 2])
            d0 = y0 - a0
            d1 = y1 - a1
            d2c = y2 - a2
            dd = d0 * d0 + d1 * d1 + d2c * d2c
            inb = (s <= j) & (j < e)
            ddm = jnp.where(inb, dd, jnp.inf)
            upd = ddm < best
            best = jnp.where(upd, ddm, best)
            bidx = jnp.where(upd, jnp.full((16,), j, jnp.int32), bidx)
            cm = jnp.min(ddm)
            j16 = jnp.full((16,), j, jnp.int32)
            cur = plsc.load_gather(colmin_v, [j16])
            plsc.store_scatter(colmin_v, [j16], jnp.minimum(cur, cm),
                               mask=lane0)
            return best, bidx

        best, bidx = inner
        b3 = bidx * 3
        xg0 = plsc.load_gather(xf_v, [b3])
        xg1 = plsc.load_gather(xf_v, [b3 + 1])
        xg2 = plsc.load_gather(xf_v, [b3 + 2])
        for yc, xg in ((y0, xg0), (y1, xg1), (y2, xg2)):
            err = yc - xg
            a = jnp.abs(err)
            hub16 = hub16 + jnp.where(a < 1.0, 0.5 * err * err, a - 0.5)
        return hub16

    hub16 = lax.fori_loop(0, _NVY, yblk, zero16)

    @plsc.parallel_loop(0, _NVSEG, unroll=4,
                        carry=(zero16, zero16, zero16))
    def dbody(k, carry):
        sp, spt, st = carry
        g = aw + k * 16 + lane
        m = (g >= alo) & (g < alo + _APW)
        z = seg_v[pl.ds(k * 16, 16)]
        p = 1.0 / (1.0 + jnp.exp(-z))
        t = aty_v[pl.ds(k * 16, 16)]
        return (sp + jnp.where(m, p, 0.0),
                spt + jnp.where(m, p * t, 0.0),
                st + jnp.where(m, t, 0.0))
    sp, spt, st = dbody

    part_v[pl.ds(0, 16)] = hub16
    part_v[pl.ds(16, 16)] = sp
    part_v[pl.ds(32, 16)] = spt
    part_v[pl.ds(48, 16)] = st
    pltpu.sync_copy(part_v, part_out.at[pl.ds(wid * 64, 64)])
    pltpu.sync_copy(colmin_v, colmin_sh.at[sid])
    plsc.subcore_barrier()

    for r in range(16):
        pltpu.sync_copy(colmin_sh.at[r, pl.ds(sid * 64, 64)],
                        mrg_v.at[pl.ds(r * 64, 64)])

    def mbody(r, carry):
        c0, c1, c2, c3 = carry
        c0 = jnp.minimum(c0, mrg_v[pl.ds(r * 64, 16)])
        c1 = jnp.minimum(c1, mrg_v[pl.ds(r * 64 + 16, 16)])
        c2 = jnp.minimum(c2, mrg_v[pl.ds(r * 64 + 32, 16)])
        c3 = jnp.minimum(c3, mrg_v[pl.ds(r * 64 + 48, 16)])
        return c0, c1, c2, c3
    cs = lax.fori_loop(0, 16, mbody, (inf16, inf16, inf16, inf16))

    y0c0 = plsc.load_gather(y016_v, [jnp.zeros((16,), jnp.int32)])
    y0c1 = plsc.load_gather(y016_v, [jnp.full((16,), 1, jnp.int32)])
    y0c2 = plsc.load_gather(y016_v, [jnp.full((16,), 2, jnp.int32)])
    for q in range(4):
        ci3 = (sid * 64 + q * 16 + lane) * 3
        g0 = plsc.load_gather(xf_v, [ci3])
        g1 = plsc.load_gather(xf_v, [ci3 + 1])
        g2 = plsc.load_gather(xf_v, [ci3 + 2])
        e0 = g0 - y0c0
        e1 = g1 - y0c1
        e2 = g2 - y0c2
        d2y0 = e0 * e0 + e1 * e1 + e2 * e2
        cq = jnp.where(jnp.isinf(cs[q]), d2y0, cs[q])
        cm_v[pl.ds(q * 16, 16)] = cq
    pltpu.sync_copy(cm_v, colmin_out.at[pl.ds(cid * _N_X + sid * 64, 64)])


def _fin_body(cm2, p2r, conf, out):
    cm = jnp.minimum(cm2[pl.ds(0, _N_X)], cm2[pl.ds(_N_X, _N_X)])
    dist = jnp.sqrt(cm)
    dc = conf[...] - dist.reshape(_N_X, 1)
    conf_loss = jnp.sum(dc * dc) / _N_X
    p2 = p2r[...]
    q = (lax.broadcasted_iota(jnp.int32, (_NW * 64,), 0) // 16) % 4
    hub = jnp.sum(jnp.where(q == 0, p2, 0.0))
    sp = jnp.sum(jnp.where(q == 1, p2, 0.0))
    spt = jnp.sum(jnp.where(q == 2, p2, 0.0))
    st = jnp.sum(jnp.where(q == 3, p2, 0.0))
    eps = 1e-6
    dice = 1.0 - (2.0 * spt + eps) / (sp + st + eps)
    out[...] = (hub / (_N_Y * 3.0) + dice + conf_loss).reshape(1, 1)


@jax.jit
def kernel(pred_seg, atom_y, pred_pos_global_node, bindingsite_center,
           preds_confidence, x_batch, y_batch):
    xf = pred_pos_global_node.reshape(-1)
    yt = bindingsite_center.T
    y3 = yt.reshape(3, _NW, _YPW).transpose(1, 0, 2)
    y0c16 = jnp.pad(yt[:, 0], (0, 13))
    seg = pred_seg.reshape(-1)

    f32 = jnp.float32
    i32 = jnp.int32
    sc = pl.kernel(
        _sc_body,
        out_type=(jax.ShapeDtypeStruct((2 * _N_X,), f32),
                  jax.ShapeDtypeStruct((_NW * 64,), f32)),
        mesh=plsc.VectorSubcoreMesh(core_axis_name="c", subcore_axis_name="s"),
        compiler_params=pltpu.CompilerParams(needs_layout_passes=False),
        scratch_types=[
            pltpu.VMEM((3 * _N_X,), f32),
            pltpu.VMEM((_N_X,), i32),
            pltpu.VMEM((_B,), i32),
            pltpu.VMEM((_B,), i32),
            pltpu.VMEM((3, _YPW), f32),
            pltpu.VMEM((_YPW,), i32),
            pltpu.VMEM((_N_X,), f32),
            pltpu.VMEM((_SEGW,), f32),
            pltpu.VMEM((_SEGW,), f32),
            pltpu.VMEM((16,), f32),
            pltpu.VMEM((64,), f32),
            pltpu.VMEM((1024,), f32),
            pltpu.VMEM((64,), f32),
            pltpu.SemaphoreType.DMA,
            pltpu.VMEM_SHARED((16, _N_X), f32),
        ],
    )
    colmin2, part2 = sc(xf, x_batch, y3, y_batch, seg, atom_y, y0c16)

    out = pl.pallas_call(
        _fin_body,
        out_shape=jax.ShapeDtypeStruct((1, 1), f32),
    )(colmin2, part2, preds_confidence)
    return out[0, 0]


# speedup vs baseline: 1.0864x; 1.0288x over previous
import jax
import jax.numpy as jnp
from jax import lax
from jax.experimental import pallas as pl
from jax.experimental.pallas import tpu as pltpu
from jax.experimental.pallas import tpu_sc as plsc

_B = 128
_N_ATOM = 100000
_N_X = 1024
_N_Y = 32768
_NW = 32
_YPW = _N_Y // _NW
_NVY = _YPW // 16
_APW = _N_ATOM // _NW
_SEGW = 3136
_NVSEG = _SEGW // 16


def _sc_body(xf_h, xb_h, y3_h, yb_h, seg_h, aty_h, y0c_h,
             colmin_out, part_out,
             xf_v, xb_v, xs_v, xe_v,
             yt_v, yb_v, colmin_v, seg_v, aty_v, y016_v,
             part_v, mrg_v, cm_v, sem,
             colmin_sh):
    cid = lax.axis_index("c")
    sid = lax.axis_index("s")
    wid = cid * 16 + sid

    alo = wid * _APW
    aw = pl.multiple_of(jnp.minimum(alo & -8, _N_ATOM - _SEGW), 8)

    cps = [
        pltpu.async_copy(xf_h, xf_v, sem),
        pltpu.async_copy(xb_h, xb_v, sem),
        pltpu.async_copy(y3_h.at[wid], yt_v, sem),
        pltpu.async_copy(yb_h.at[pl.ds(wid * _YPW, _YPW)], yb_v, sem),
        pltpu.async_copy(seg_h.at[pl.ds(aw, _SEGW)], seg_v, sem),
        pltpu.async_copy(aty_h.at[pl.ds(aw, _SEGW)], aty_v, sem),
        pltpu.async_copy(y0c_h, y016_v, sem),
    ]

    inf16 = jnp.full((16,), jnp.inf, jnp.float32)
    zero16 = jnp.zeros((16,), jnp.float32)
    lane = lax.iota(jnp.int32, 16)
    lane0 = lane == 0

    @plsc.parallel_loop(0, _N_X // 16, unroll=4)
    def _init_body(i):
        colmin_v[pl.ds(i * 16, 16)] = inf16

    for cp in cps:
        cp.wait()

    @plsc.parallel_loop(0, _B // 16, unroll=2)
    def bs_body(t):
        b = lane + t * 16
        lo = jnp.zeros((16,), jnp.int32)
        hi = jnp.full((16,), _N_X, jnp.int32)
        lo2 = jnp.zeros((16,), jnp.int32)
        hi2 = jnp.full((16,), _N_X, jnp.int32)
        for _unused in range(11):
            mid = jnp.minimum((lo + hi) >> 1, _N_X - 1)
            v = plsc.load_gather(xb_v, [mid])
            p = v < b
            lo = jnp.where(p, mid + 1, lo)
            hi = jnp.where(p, hi, mid)
            mid2 = jnp.minimum((lo2 + hi2) >> 1, _N_X - 1)
            v2 = plsc.load_gather(xb_v, [mid2])
            p2 = v2 <= b
            lo2 = jnp.where(p2, mid2 + 1, lo2)
            hi2 = jnp.where(p2, hi2, mid2)
        xs_v[pl.ds(t * 16, 16)] = lo
        xe_v[pl.ds(t * 16, 16)] = lo2

    def yblk(k, hub16):
        yb16 = yb_v[pl.ds(k * 16, 16)]
        y0 = yt_v[0, pl.ds(k * 16, 16)]
        y1 = yt_v[1, pl.ds(k * 16, 16)]
        y2 = yt_v[2, pl.ds(k * 16, 16)]
        s = plsc.load_gather(xs_v, [yb16])
        e = plsc.load_gather(xe_v, [yb16])
        jlo = jnp.min(s)
        jhi = jnp.max(e)

        @plsc.parallel_loop(jlo, jhi, unroll=2, carry=(inf16, jnp.zeros((16,), jnp.int32)))
        def inner(j, carry):
            best, bidx = carry
            j3 = jnp.full((16,), j * 3, jnp.int32)
            a0 = plsc.load_gather(xf_v, [j3])
            a1 = plsc.load_gather(xf_v, [j3 + 1])
            a2 = plsc.load_gather(xf_v, [j3 + 2])
            d0 = y0 - a0
            d1 = y1 - a1
            d2c = y2 - a2
            dd = d0 * d0 + d1 * d1 + d2c * d2c
            inb = (s <= j) & (j < e)
            ddm = jnp.where(inb, dd, jnp.inf)
            upd = ddm < best
            best = jnp.where(upd, ddm, best)
            bidx = jnp.where(upd, jnp.full((16,), j, jnp.int32), bidx)
            cm = jnp.min(ddm)
            j16 = jnp.full((16,), j, jnp.int32)
            cur = plsc.load_gather(colmin_v, [j16])
            plsc.store_scatter(colmin_v, [j16], jnp.minimum(cur, cm),
                               mask=lane0)
            return best, bidx

        best, bidx = inner
        b3 = bidx * 3
        xg0 = plsc.load_gather(xf_v, [b3])
        xg1 = plsc.load_gather(xf_v, [b3 + 1])
        xg2 = plsc.load_gather(xf_v, [b3 + 2])
        for yc, xg in ((y0, xg0), (y1, xg1), (y2, xg2)):
            err = yc - xg
            a = jnp.abs(err)
            hub16 = hub16 + jnp.where(a < 1.0, 0.5 * err * err, a - 0.5)
        return hub16

    hub16 = lax.fori_loop(0, _NVY, yblk, zero16)

    @plsc.parallel_loop(0, _NVSEG, unroll=4,
                        carry=(zero16, zero16, zero16))
    def dbody(k, carry):
        sp, spt, st = carry
        g = aw + k * 16 + lane
        m = (g >= alo) & (g < alo + _APW)
        z = seg_v[pl.ds(k * 16, 16)]
        p = 1.0 / (1.0 + jnp.exp(-z))
        t = aty_v[pl.ds(k * 16, 16)]
        return (sp + jnp.where(m, p, 0.0),
                spt + jnp.where(m, p * t, 0.0),
                st + jnp.where(m, t, 0.0))
    sp, spt, st = dbody

    part_v[pl.ds(0, 16)] = hub16
    part_v[pl.ds(16, 16)] = sp
    part_v[pl.ds(32, 16)] = spt
    part_v[pl.ds(48, 16)] = st
    pltpu.sync_copy(part_v, part_out.at[pl.ds(wid * 64, 64)])
    pltpu.sync_copy(colmin_v, colmin_sh.at[sid])
    plsc.subcore_barrier()

    for r in range(16):
        pltpu.sync_copy(colmin_sh.at[r, pl.ds(sid * 64, 64)],
                        mrg_v.at[pl.ds(r * 64, 64)])

    def mbody(r, carry):
        c0, c1, c2, c3 = carry
        c0 = jnp.minimum(c0, mrg_v[pl.ds(r * 64, 16)])
        c1 = jnp.minimum(c1, mrg_v[pl.ds(r * 64 + 16, 16)])
        c2 = jnp.minimum(c2, mrg_v[pl.ds(r * 64 + 32, 16)])
        c3 = jnp.minimum(c3, mrg_v[pl.ds(r * 64 + 48, 16)])
        return c0, c1, c2, c3
    cs = lax.fori_loop(0, 16, mbody, (inf16, inf16, inf16, inf16))

    y0c0 = plsc.load_gather(y016_v, [jnp.zeros((16,), jnp.int32)])
    y0c1 = plsc.load_gather(y016_v, [jnp.full((16,), 1, jnp.int32)])
    y0c2 = plsc.load_gather(y016_v, [jnp.full((16,), 2, jnp.int32)])
    for q in range(4):
        ci3 = (sid * 64 + q * 16 + lane) * 3
        g0 = plsc.load_gather(xf_v, [ci3])
        g1 = plsc.load_gather(xf_v, [ci3 + 1])
        g2 = plsc.load_gather(xf_v, [ci3 + 2])
        e0 = g0 - y0c0
        e1 = g1 - y0c1
        e2 = g2 - y0c2
        d2y0 = e0 * e0 + e1 * e1 + e2 * e2
        cq = jnp.where(jnp.isinf(cs[q]), d2y0, cs[q])
        cm_v[pl.ds(q * 16, 16)] = cq
    pltpu.sync_copy(cm_v, colmin_out.at[pl.ds(cid * _N_X + sid * 64, 64)])


def _fin_body(cm2, p2r, conf, out):
    cm = jnp.minimum(cm2[pl.ds(0, _N_X)], cm2[pl.ds(_N_X, _N_X)])
    dist = jnp.sqrt(cm)
    dc = conf[...] - dist.reshape(_N_X, 1)
    conf_loss = jnp.sum(dc * dc) / _N_X
    p2 = p2r[...]
    q = (lax.broadcasted_iota(jnp.int32, (_NW * 64,), 0) // 16) % 4
    hub = jnp.sum(jnp.where(q == 0, p2, 0.0))
    sp = jnp.sum(jnp.where(q == 1, p2, 0.0))
    spt = jnp.sum(jnp.where(q == 2, p2, 0.0))
    st = jnp.sum(jnp.where(q == 3, p2, 0.0))
    eps = 1e-6
    dice = 1.0 - (2.0 * spt + eps) / (sp + st + eps)
    out[...] = (hub / (_N_Y * 3.0) + dice + conf_loss).reshape(1, 1)


@jax.jit
def kernel(pred_seg, atom_y, pred_pos_global_node, bindingsite_center,
           preds_confidence, x_batch, y_batch):
    xf = pred_pos_global_node.reshape(-1)
    yt = bindingsite_center.T
    y3 = yt.reshape(3, _NW, _YPW).transpose(1, 0, 2)
    y0c16 = jnp.pad(yt[:, 0], (0, 13))
    seg = pred_seg.reshape(-1)

    f32 = jnp.float32
    i32 = jnp.int32
    sc = pl.kernel(
        _sc_body,
        out_type=(jax.ShapeDtypeStruct((2 * _N_X,), f32),
                  jax.ShapeDtypeStruct((_NW * 64,), f32)),
        mesh=plsc.VectorSubcoreMesh(core_axis_name="c", subcore_axis_name="s"),
        compiler_params=pltpu.CompilerParams(needs_layout_passes=False),
        scratch_types=[
            pltpu.VMEM((3 * _N_X,), f32),
            pltpu.VMEM((_N_X,), i32),
            pltpu.VMEM((_B,), i32),
            pltpu.VMEM((_B,), i32),
            pltpu.VMEM((3, _YPW), f32),
            pltpu.VMEM((_YPW,), i32),
            pltpu.VMEM((_N_X,), f32),
            pltpu.VMEM((_SEGW,), f32),
            pltpu.VMEM((_SEGW,), f32),
            pltpu.VMEM((16,), f32),
            pltpu.VMEM((64,), f32),
            pltpu.VMEM((1024,), f32),
            pltpu.VMEM((64,), f32),
            pltpu.SemaphoreType.DMA,
            pltpu.VMEM_SHARED((16, _N_X), f32),
        ],
    )
    colmin2, part2 = sc(xf, x_batch, y3, y_batch, seg, atom_y, y0c16)

    out = pl.pallas_call(
        _fin_body,
        out_shape=jax.ShapeDtypeStruct((1, 1), f32),
    )(colmin2, part2, preds_confidence)
    return out[0, 0]
